# Initial kernel scaffold; baseline (speedup 1.0000x reference)
import jax
import jax.numpy as jnp
from jax.experimental import pallas as pl


def kernel(x_querry, l, x_block, e_k, e_p):
    raise NotImplementedError("write your pallas kernel here")


# speedup vs baseline: 1.0540x; 1.0540x over previous
import functools

import jax
import jax.numpy as jnp
from jax import lax
from jax.experimental import pallas as pl
from jax.experimental.pallas import tpu as pltpu
from jax.experimental.pallas import tpu_sc as plsc

_NC = 2
_NS = 16
_NW = _NC * _NS


def _topk_body(q_ref, nk_ref, idxk_ref, idxv_ref):
    scores = lax.dot_general(q_ref[...], nk_ref[...], (((1,), (1,)), ((), ())),
                             preferred_element_type=jnp.float32)
    m = jnp.max(scores, axis=1, keepdims=True)
    col = lax.broadcasted_iota(jnp.int32, scores.shape, 1)
    amax = jnp.min(jnp.where(scores == m, col, jnp.int32(2**30)), axis=1)
    idxk_ref[...] = amax * 2
    idxv_ref[...] = amax * 2 + 1


def _topk_indices(q, nk):
    b = q.shape[0]
    return pl.pallas_call(
        _topk_body,
        out_shape=(jax.ShapeDtypeStruct((b,), jnp.int32),
                   jax.ShapeDtypeStruct((b,), jnp.int32)),
    )(q, nk)


def _make_gather(b, d, rows):
    bpw = b // _NW
    chunk = 8
    nstep = 2 * (bpw // chunk)
    mesh = plsc.VectorSubcoreMesh(core_axis_name="c", subcore_axis_name="s")

    @functools.partial(
        pl.kernel,
        mesh=mesh,
        out_type=(jax.ShapeDtypeStruct((b, d), jnp.float32),
                  jax.ShapeDtypeStruct((b, d), jnp.float32)),
        scratch_types=[
            pltpu.VMEM((bpw,), jnp.int32),
            pltpu.VMEM((bpw,), jnp.int32),
            pltpu.VMEM((chunk, d), jnp.float32),
            pltpu.VMEM((chunk, d), jnp.float32),
            pltpu.SemaphoreType.DMA,
            pltpu.SemaphoreType.DMA,
            pltpu.SemaphoreType.DMA,
            pltpu.SemaphoreType.DMA,
        ],
    )
    def gather_kernel(table_hbm, idxk_hbm, idxv_hbm, ek_out, ev_out,
                      idxk_v, idxv_v, b0, b1, sg0, sg1, sw0, sw1):
        c = lax.axis_index("c")
        s = lax.axis_index("s")
        wid = s * _NC + c
        base = wid * bpw

        pltpu.sync_copy(idxk_hbm.at[pl.ds(base, bpw)], idxk_v)
        pltpu.sync_copy(idxv_hbm.at[pl.ds(base, bpw)], idxv_v)

        buf = (b0, b1)
        sgat = (sg0, sg1)
        swrt = (sw0, sw1)
        gat = {}
        wrt = {}

        def start_gather(t):
            idx = idxk_v if t % 2 == 0 else idxv_v
            off = (t // 2) * chunk
            gat[t] = pltpu.async_copy(
                table_hbm.at[idx.at[pl.ds(off, chunk)]], buf[t % 2],
                sgat[t % 2])

        def start_write(t):
            out = ek_out if t % 2 == 0 else ev_out
            off = base + (t // 2) * chunk
            wrt[t] = pltpu.async_copy(
                buf[t % 2], out.at[pl.ds(off, chunk)], swrt[t % 2])

        start_gather(0)
        for t in range(nstep):
            if t + 1 < nstep:
                if t >= 1:
                    wrt[t - 1].wait()
                start_gather(t + 1)
            gat[t].wait()
            start_write(t)
        wrt[nstep - 2].wait()
        wrt[nstep - 1].wait()

    return gather_kernel


def kernel(x_querry, l, x_block, e_k, e_p):
    b = x_querry.shape[0]
    pool, plen, emb = e_p.shape
    half = plen // 2
    d = half * emb

    nk = e_k / jnp.maximum(jnp.linalg.norm(e_k, axis=1, keepdims=True), 1e-12)
    q = x_querry / jnp.maximum(
        jnp.linalg.norm(x_querry, axis=1, keepdims=True), 1e-12)
    idxk, idxv = _topk_indices(q, nk)
    table = e_p.reshape(2 * pool, d)
    ekf, evf = _make_gather(b, d, 2 * pool)(table, idxk, idxv)
    return (ekf.reshape(b, half, emb), evf.reshape(b, half, emb), x_block)


# speedup vs baseline: 1.9675x; 1.8667x over previous
import functools

import jax
import jax.numpy as jnp
from jax import lax
from jax.experimental import pallas as pl
from jax.experimental.pallas import tpu as pltpu
from jax.experimental.pallas import tpu_sc as plsc

_NC = 2
_NS = 16
_NW = _NC * _NS


def _topk_body(q_ref, nk_ref, idxk_ref, idxv_ref):
    scores = lax.dot_general(q_ref[...], nk_ref[...], (((1,), (1,)), ((), ())),
                             preferred_element_type=jnp.float32)
    m = jnp.max(scores, axis=1, keepdims=True)
    col = lax.broadcasted_iota(jnp.int32, scores.shape, 1)
    amax = jnp.min(jnp.where(scores == m, col, jnp.int32(2**30)), axis=1)
    idxk_ref[...] = amax * 2
    idxv_ref[...] = amax * 2 + 1


def _topk_indices(q, nk):
    b = q.shape[0]
    return pl.pallas_call(
        _topk_body,
        out_shape=(jax.ShapeDtypeStruct((b,), jnp.int32),
                   jax.ShapeDtypeStruct((b,), jnp.int32)),
    )(q, nk)


def _make_gather(b, half, emb, rows):
    bpw = b // _NW
    chunk = 8
    nstep = 2 * (bpw // chunk)
    mesh = plsc.VectorSubcoreMesh(core_axis_name="c", subcore_axis_name="s")

    @functools.partial(
        pl.kernel,
        mesh=mesh,
        out_type=(jax.ShapeDtypeStruct((b, half, emb), jnp.float32),
                  jax.ShapeDtypeStruct((b, half, emb), jnp.float32)),
        scratch_types=[
            pltpu.VMEM((bpw,), jnp.int32),
            pltpu.VMEM((bpw,), jnp.int32),
            pltpu.VMEM((chunk, half, emb), jnp.float32),
            pltpu.VMEM((chunk, half, emb), jnp.float32),
            pltpu.SemaphoreType.DMA,
            pltpu.SemaphoreType.DMA,
            pltpu.SemaphoreType.DMA,
            pltpu.SemaphoreType.DMA,
        ],
    )
    def gather_kernel(table_hbm, idxk_hbm, idxv_hbm, ek_out, ev_out,
                      idxk_v, idxv_v, b0, b1, sg0, sg1, sw0, sw1):
        c = lax.axis_index("c")
        s = lax.axis_index("s")
        wid = s * _NC + c
        base = wid * bpw

        pltpu.sync_copy(idxk_hbm.at[pl.ds(base, bpw)], idxk_v)
        pltpu.sync_copy(idxv_hbm.at[pl.ds(base, bpw)], idxv_v)

        buf = (b0, b1)
        sgat = (sg0, sg1)
        swrt = (sw0, sw1)
        gat = {}
        wrt = {}

        def start_gather(t):
            idx = idxk_v if t % 2 == 0 else idxv_v
            off = (t // 2) * chunk
            gat[t] = pltpu.async_copy(
                table_hbm.at[idx.at[pl.ds(off, chunk)]], buf[t % 2],
                sgat[t % 2])

        def start_write(t):
            out = ek_out if t % 2 == 0 else ev_out
            off = base + (t // 2) * chunk
            wrt[t] = pltpu.async_copy(
                buf[t % 2], out.at[pl.ds(off, chunk)], swrt[t % 2])

        start_gather(0)
        for t in range(nstep):
            if t + 1 < nstep:
                if t >= 1:
                    wrt[t - 1].wait()
                start_gather(t + 1)
            gat[t].wait()
            start_write(t)
        wrt[nstep - 2].wait()
        wrt[nstep - 1].wait()

    return gather_kernel


def kernel(x_querry, l, x_block, e_k, e_p):
    b = x_querry.shape[0]
    pool, plen, emb = e_p.shape
    half = plen // 2
    d = half * emb

    nk = e_k / jnp.maximum(jnp.linalg.norm(e_k, axis=1, keepdims=True), 1e-12)
    q = x_querry / jnp.maximum(
        jnp.linalg.norm(x_querry, axis=1, keepdims=True), 1e-12)
    idxk, idxv = _topk_indices(q, nk)
    table = e_p.reshape(2 * pool, half, emb)
    ek_o, ev_o = _make_gather(b, half, emb, 2 * pool)(table, idxk, idxv)
    return (ek_o, ev_o, x_block)


# speedup vs baseline: 2.3951x; 1.2173x over previous
import functools

import jax
import jax.numpy as jnp
from jax import lax
from jax.experimental import pallas as pl
from jax.experimental.pallas import tpu as pltpu
from jax.experimental.pallas import tpu_sc as plsc

_NC = 2
_NS = 16
_NW = _NC * _NS


def _topk_body(q_ref, nk_ref, idxk_ref, idxv_ref):
    scores = lax.dot_general(q_ref[...], nk_ref[...], (((1,), (1,)), ((), ())),
                             preferred_element_type=jnp.float32)
    m = jnp.max(scores, axis=1, keepdims=True)
    col = lax.broadcasted_iota(jnp.int32, scores.shape, 1)
    amax = jnp.min(jnp.where(scores == m, col, jnp.int32(2**30)), axis=1)
    idxk_ref[...] = amax * 2
    idxv_ref[...] = amax * 2 + 1


def _topk_indices(q, nk):
    b = q.shape[0]
    return pl.pallas_call(
        _topk_body,
        out_shape=(jax.ShapeDtypeStruct((b,), jnp.int32),
                   jax.ShapeDtypeStruct((b,), jnp.int32)),
    )(q, nk)


def _make_gather(b, half, emb, rows):
    bpw = b // _NW
    chunk = 8
    nstep = 2 * (bpw // chunk)
    mesh = plsc.VectorSubcoreMesh(core_axis_name="c", subcore_axis_name="s")

    @functools.partial(
        pl.kernel,
        mesh=mesh,
        out_type=(jax.ShapeDtypeStruct((b, half, emb), jnp.float32),
                  jax.ShapeDtypeStruct((b, half, emb), jnp.float32)),
        scratch_types=[
            pltpu.VMEM((bpw,), jnp.int32),
            pltpu.VMEM((bpw,), jnp.int32),
            pltpu.VMEM((chunk, half, emb), jnp.float32),
            pltpu.VMEM((chunk, half, emb), jnp.float32),
            pltpu.SemaphoreType.DMA,
            pltpu.SemaphoreType.DMA,
            pltpu.SemaphoreType.DMA,
            pltpu.SemaphoreType.DMA,
        ],
    )
    def gather_kernel(table_hbm, idxk_hbm, idxv_hbm, ek_out, ev_out,
                      idxk_v, idxv_v, b0, b1, sg0, sg1, sw0, sw1):
        c = lax.axis_index("c")
        s = lax.axis_index("s")
        wid = s * _NC + c
        base = wid * bpw

        pltpu.sync_copy(idxk_hbm.at[pl.ds(base, bpw)], idxk_v)
        pltpu.sync_copy(idxv_hbm.at[pl.ds(base, bpw)], idxv_v)

        buf = (b0, b1)
        sgat = (sg0, sg1)
        swrt = (sw0, sw1)
        gat = {}
        wrt = {}

        def start_gather(t):
            idx = idxk_v if t % 2 == 0 else idxv_v
            off = (t // 2) * chunk
            gat[t] = pltpu.async_copy(
                table_hbm.at[idx.at[pl.ds(off, chunk)]], buf[t % 2],
                sgat[t % 2])

        def start_write(t):
            out = ek_out if t % 2 == 0 else ev_out
            off = base + (t // 2) * chunk
            wrt[t] = pltpu.async_copy(
                buf[t % 2], out.at[pl.ds(off, chunk)], swrt[t % 2])

        start_gather(0)
        for t in range(nstep):
            if t + 1 < nstep:
                if t >= 1:
                    wrt[t - 1].wait()
                start_gather(t + 1)
            gat[t].wait()
            start_write(t)
        wrt[nstep - 2].wait()
        wrt[nstep - 1].wait()

    return gather_kernel


def _make_gather_spmem(b, half, emb, rows):
    bpw = b // _NW
    mesh = plsc.VectorSubcoreMesh(core_axis_name="c", subcore_axis_name="s")

    @functools.partial(
        pl.kernel,
        mesh=mesh,
        out_type=(jax.ShapeDtypeStruct((b, half, emb), jnp.float32),
                  jax.ShapeDtypeStruct((b, half, emb), jnp.float32)),
        scratch_types=[
            pltpu.VMEM((bpw,), jnp.int32),
            pltpu.VMEM_SHARED((rows, half, emb), jnp.float32),
            pltpu.SemaphoreType.DMA,
        ],
    )
    def gather_kernel(table_hbm, idxk_hbm, ek_out, ev_out, idx_v, shared,
                      sem):
        c = lax.axis_index("c")
        s = lax.axis_index("s")
        wid = s * _NC + c
        base = wid * bpw

        @pl.when(s == 0)
        def _():
            pltpu.sync_copy(table_hbm, shared)

        pltpu.sync_copy(idxk_hbm.at[pl.ds(base, bpw)], idx_v)
        plsc.subcore_barrier()

        def body(g, carry):
            off = pl.multiple_of(g * 16, 16)
            vec = idx_v[pl.ds(off, 16)]
            for j in range(16):
                rk = vec[j]
                i = off + j
                pltpu.async_copy(shared.at[pl.ds(rk, 1)],
                                 ek_out.at[pl.ds(base + i, 1)], sem)
                pltpu.async_copy(shared.at[pl.ds(rk + 1, 1)],
                                 ev_out.at[pl.ds(base + i, 1)], sem)
            return carry

        lax.fori_loop(0, bpw // 16, body, 0)
        pltpu.make_async_copy(ek_out.at[pl.ds(base, bpw)],
                              ek_out.at[pl.ds(base, bpw)], sem).wait()
        pltpu.make_async_copy(ev_out.at[pl.ds(base, bpw)],
                              ev_out.at[pl.ds(base, bpw)], sem).wait()

    return gather_kernel


def kernel(x_querry, l, x_block, e_k, e_p):
    b = x_querry.shape[0]
    pool, plen, emb = e_p.shape
    half = plen // 2
    d = half * emb

    nk = e_k / jnp.maximum(jnp.linalg.norm(e_k, axis=1, keepdims=True), 1e-12)
    q = x_querry / jnp.maximum(
        jnp.linalg.norm(x_querry, axis=1, keepdims=True), 1e-12)
    idxk, idxv = _topk_indices(q, nk)
    table = e_p.reshape(2 * pool, half, emb)
    ek_o, ev_o = _make_gather_spmem(b, half, emb, 2 * pool)(table, idxk)
    return (ek_o, ev_o, x_block)


# speedup vs baseline: 2.6542x; 1.1082x over previous
import functools

import jax
import jax.numpy as jnp
from jax import lax
from jax.experimental import pallas as pl
from jax.experimental.pallas import tpu as pltpu
from jax.experimental.pallas import tpu_sc as plsc

_NC = 2
_NS = 16
_NW = _NC * _NS


def _topk_body(x_ref, nk_ref, idxk_ref, idxv_ref):
    x = x_ref[...]
    q = x / jnp.maximum(jnp.sqrt(jnp.sum(x * x, axis=1, keepdims=True)),
                        1e-12)
    scores = lax.dot_general(q, nk_ref[...], (((1,), (1,)), ((), ())),
                             preferred_element_type=jnp.float32)
    m = jnp.max(scores, axis=1, keepdims=True)
    col = lax.broadcasted_iota(jnp.int32, scores.shape, 1)
    amax = jnp.min(jnp.where(scores == m, col, jnp.int32(2**30)), axis=1)
    idxk_ref[...] = amax * 2
    idxv_ref[...] = amax * 2 + 1


def _topk_indices(x, nk):
    b = x.shape[0]
    return pl.pallas_call(
        _topk_body,
        out_shape=(jax.ShapeDtypeStruct((b,), jnp.int32),
                   jax.ShapeDtypeStruct((b,), jnp.int32)),
    )(x, nk)


def _make_gather(b, half, emb, rows):
    bpw = b // _NW
    chunk = 8
    nstep = 2 * (bpw // chunk)
    mesh = plsc.VectorSubcoreMesh(core_axis_name="c", subcore_axis_name="s")

    @functools.partial(
        pl.kernel,
        mesh=mesh,
        out_type=(jax.ShapeDtypeStruct((b, half, emb), jnp.float32),
                  jax.ShapeDtypeStruct((b, half, emb), jnp.float32)),
        scratch_types=[
            pltpu.VMEM((bpw,), jnp.int32),
            pltpu.VMEM((bpw,), jnp.int32),
            pltpu.VMEM((chunk, half, emb), jnp.float32),
            pltpu.VMEM((chunk, half, emb), jnp.float32),
            pltpu.SemaphoreType.DMA,
            pltpu.SemaphoreType.DMA,
            pltpu.SemaphoreType.DMA,
            pltpu.SemaphoreType.DMA,
        ],
    )
    def gather_kernel(table_hbm, idxk_hbm, idxv_hbm, ek_out, ev_out,
                      idxk_v, idxv_v, b0, b1, sg0, sg1, sw0, sw1):
        c = lax.axis_index("c")
        s = lax.axis_index("s")
        wid = s * _NC + c
        base = wid * bpw

        pltpu.sync_copy(idxk_hbm.at[pl.ds(base, bpw)], idxk_v)
        pltpu.sync_copy(idxv_hbm.at[pl.ds(base, bpw)], idxv_v)

        buf = (b0, b1)
        sgat = (sg0, sg1)
        swrt = (sw0, sw1)
        gat = {}
        wrt = {}

        def start_gather(t):
            idx = idxk_v if t % 2 == 0 else idxv_v
            off = (t // 2) * chunk
            gat[t] = pltpu.async_copy(
                table_hbm.at[idx.at[pl.ds(off, chunk)]], buf[t % 2],
                sgat[t % 2])

        def start_write(t):
            out = ek_out if t % 2 == 0 else ev_out
            off = base + (t // 2) * chunk
            wrt[t] = pltpu.async_copy(
                buf[t % 2], out.at[pl.ds(off, chunk)], swrt[t % 2])

        start_gather(0)
        for t in range(nstep):
            if t + 1 < nstep:
                if t >= 1:
                    wrt[t - 1].wait()
                start_gather(t + 1)
            gat[t].wait()
            start_write(t)
        wrt[nstep - 2].wait()
        wrt[nstep - 1].wait()

    return gather_kernel


def _make_gather_spmem(b, half, emb, rows):
    bpw = b // _NW
    mesh = plsc.VectorSubcoreMesh(core_axis_name="c", subcore_axis_name="s")

    @functools.partial(
        pl.kernel,
        mesh=mesh,
        out_type=(jax.ShapeDtypeStruct((b, half, emb), jnp.float32),
                  jax.ShapeDtypeStruct((b, half, emb), jnp.float32)),
        scratch_types=[
            pltpu.VMEM((bpw,), jnp.int32),
            pltpu.VMEM_SHARED((rows, half, emb), jnp.float32),
            pltpu.SemaphoreType.DMA,
        ],
    )
    def gather_kernel(table_hbm, idxk_hbm, ek_out, ev_out, idx_v, shared,
                      sem):
        c = lax.axis_index("c")
        s = lax.axis_index("s")
        wid = s * _NC + c
        base = wid * bpw

        @pl.when(s == 0)
        def _():
            pltpu.sync_copy(table_hbm, shared)

        pltpu.sync_copy(idxk_hbm.at[pl.ds(base, bpw)], idx_v)
        plsc.subcore_barrier()

        def body(g, carry):
            off = pl.multiple_of(g * 16, 16)
            vec = idx_v[pl.ds(off, 16)]
            for j in range(16):
                rk = vec[j]
                i = off + j
                pltpu.async_copy(shared.at[pl.ds(rk, 1)],
                                 ek_out.at[pl.ds(base + i, 1)], sem)
                pltpu.async_copy(shared.at[pl.ds(rk + 1, 1)],
                                 ev_out.at[pl.ds(base + i, 1)], sem)
            return carry

        lax.fori_loop(0, bpw // 16, body, 0)
        pltpu.make_async_copy(ek_out.at[pl.ds(base, bpw)],
                              ek_out.at[pl.ds(base, bpw)], sem).wait()
        pltpu.make_async_copy(ev_out.at[pl.ds(base, bpw)],
                              ev_out.at[pl.ds(base, bpw)], sem).wait()

    return gather_kernel


def kernel(x_querry, l, x_block, e_k, e_p):
    b = x_querry.shape[0]
    pool, plen, emb = e_p.shape
    half = plen // 2
    d = half * emb

    nk = e_k / jnp.maximum(jnp.linalg.norm(e_k, axis=1, keepdims=True), 1e-12)
    idxk, idxv = _topk_indices(x_querry, nk)
    table = e_p.reshape(2 * pool, half, emb)
    ek_o, ev_o = _make_gather_spmem(b, half, emb, 2 * pool)(table, idxk)
    return (ek_o, ev_o, x_block)


# speedup vs baseline: 3.4329x; 1.2934x over previous
import functools

import jax
import jax.numpy as jnp
from jax import lax
from jax.experimental import pallas as pl
from jax.experimental.pallas import tpu as pltpu
from jax.experimental.pallas import tpu_sc as plsc

_NC = 2
_NS = 16
_NW = _NC * _NS


def _topk_body(x_ref, nk_ref, idx_ref):
    x = x_ref[...]
    q = x / jnp.maximum(jnp.sqrt(jnp.sum(x * x, axis=1, keepdims=True)),
                        1e-12)
    scores = lax.dot_general(q, nk_ref[...], (((1,), (1,)), ((), ())),
                             preferred_element_type=jnp.float32)
    m = jnp.max(scores, axis=1, keepdims=True)
    col = lax.broadcasted_iota(jnp.int32, scores.shape, 1)
    idx_ref[...] = jnp.min(jnp.where(scores == m, col, jnp.int32(2**30)),
                           axis=1)


def _topk_indices(x, nk):
    b = x.shape[0]
    return pl.pallas_call(
        _topk_body,
        out_shape=jax.ShapeDtypeStruct((b,), jnp.int32),
    )(x, nk)


def _make_gather_spmem(b, half, emb, rows):
    bpw = b // _NW
    mesh = plsc.VectorSubcoreMesh(core_axis_name="c", subcore_axis_name="s")

    @functools.partial(
        pl.kernel,
        mesh=mesh,
        out_type=jax.ShapeDtypeStruct((b, half, emb), jnp.float32),
        scratch_types=[
            pltpu.VMEM((bpw,), jnp.int32),
            pltpu.VMEM_SHARED((rows, half, emb), jnp.float32),
            pltpu.SemaphoreType.DMA,
        ],
    )
    def gather_kernel(table_hbm, idxk_hbm, ek_out, idx_v, shared, sem):
        c = lax.axis_index("c")
        s = lax.axis_index("s")
        wid = s * _NC + c
        base = wid * bpw

        @pl.when(s == 0)
        def _():
            pltpu.sync_copy(table_hbm, shared)

        pltpu.sync_copy(idxk_hbm.at[pl.ds(base, bpw)], idx_v)
        plsc.subcore_barrier()

        def body(g, carry):
            off = pl.multiple_of(g * 16, 16)
            vec = idx_v[pl.ds(off, 16)]
            for j in range(16):
                pltpu.async_copy(shared.at[pl.ds(vec[j], 1)],
                                 ek_out.at[pl.ds(base + off + j, 1)], sem)
            return carry

        lax.fori_loop(0, bpw // 16, body, 0)
        pltpu.make_async_copy(ek_out.at[pl.ds(base, bpw)],
                              ek_out.at[pl.ds(base, bpw)], sem).wait()

    return gather_kernel


def _make_gather_tc(b, half, emb):
    blk = 512

    def body(idx_ref, tab_ref, out_ref):
        amax = idx_ref[...]
        col = lax.broadcasted_iota(jnp.int32, (blk, 128), 1)
        oh = (col == amax[:, None]).astype(jnp.float32)
        for h in range(half):
            out_ref[:, h, :] = lax.dot_general(
                oh, tab_ref[:, h, :], (((1,), (0,)), ((), ())),
                preferred_element_type=jnp.float32)

    return pl.pallas_call(
        body,
        grid=(b // blk,),
        in_specs=[pl.BlockSpec((blk,), lambda g: (g,)),
                  pl.BlockSpec((128, half, emb), lambda g: (0, 0, 0))],
        out_specs=pl.BlockSpec((blk, half, emb), lambda g: (g, 0, 0)),
        out_shape=jax.ShapeDtypeStruct((b, half, emb), jnp.float32),
    )


def kernel(x_querry, l, x_block, e_k, e_p):
    b = x_querry.shape[0]
    pool, plen, emb = e_p.shape
    half = plen // 2
    d = half * emb

    nk = e_k / jnp.maximum(jnp.linalg.norm(e_k, axis=1, keepdims=True), 1e-12)
    amax = _topk_indices(x_querry, nk)
    tab_k = e_p[:, :half, :]
    tab_v = jnp.zeros((128, half, emb), jnp.float32).at[:pool].set(
        e_p[:, half:, :])
    ek_o = _make_gather_spmem(b, half, emb, pool)(tab_k, amax)
    ev_o = _make_gather_tc(b, half, emb)(amax, tab_v)
    return (ek_o, ev_o, x_block)
